# Initial kernel scaffold; baseline (speedup 1.0000x reference)
import jax
import jax.numpy as jnp
from jax.experimental import pallas as pl


def kernel(x, edge_index, sh, edge_attr, batch, W_atom, b_atom, W_sh1, b_sh1, W_sh2, b_sh2, W_r1, b_r1, W_r2, b_r2, Wm, bm, We, be, Wu, bu, W_f1, b_f1, W_f2, b_f2):
    raise NotImplementedError("write your pallas kernel here")


# speedup vs baseline: 2.7094x; 2.7094x over previous
import functools

import jax
import jax.numpy as jnp
from jax import lax
from jax.experimental import pallas as pl
from jax.experimental.pallas import tpu as pltpu
from jax.experimental.pallas import tpu_sc as plsc

N = 50000
E = 800000
IN_DIM = 92
NF = 32
DIST_DIM = 16
SH_DIM = 16
SH_L2 = 9
NUM_LAYERS = 3
NUM_GRAPHS = 64
RBF_BINS = 64

BN = 1024
N_PAD = 50176
N_BLOCKS = N_PAD // BN
BE = 2048
NW = 32
BLK = 128
NBLK = 196
EPW = BLK * NBLK
E_PAD = NW * EPW
E_BLOCKS = E_PAD // BE
STRIPE = N_PAD // 16
ZR = 196
NZ = STRIPE // ZR


def _silu(v):
    return v * jax.nn.sigmoid(v)


def _embed_body(x_ref, wa_ref, ba_ref, wm0_ref, bm0_ref, h_ref, hm_ref):
    h = jnp.dot(x_ref[...], wa_ref[...],
                preferred_element_type=jnp.float32) + ba_ref[...]
    h_ref[...] = h
    hm_ref[...] = jnp.dot(h, wm0_ref[...],
                          preferred_element_type=jnp.float32) + bm0_ref[...]


def _embed(x_p, wa, ba, wm0, bm0):
    return pl.pallas_call(
        _embed_body,
        grid=(N_BLOCKS,),
        in_specs=[
            pl.BlockSpec((BN, IN_DIM), lambda i: (i, 0)),
            pl.BlockSpec((IN_DIM, NF), lambda i: (0, 0)),
            pl.BlockSpec((1, NF), lambda i: (0, 0)),
            pl.BlockSpec((NF, NF), lambda i: (0, 0)),
            pl.BlockSpec((1, NF), lambda i: (0, 0)),
        ],
        out_specs=[
            pl.BlockSpec((BN, NF), lambda i: (i, 0)),
            pl.BlockSpec((BN, NF), lambda i: (i, 0)),
        ],
        out_shape=[
            jax.ShapeDtypeStruct((N_PAD, NF), jnp.float32),
            jax.ShapeDtypeStruct((N_PAD, NF), jnp.float32),
        ],
    )(x_p, wa, ba, wm0, bm0)


def _gates_body(sh_ref, ea_ref, cen_ref, gam_ref, wr1_ref, br1_ref, wr2_ref,
                br2_ref, ws1_ref, bs1_ref, ws2_ref, bs2_ref, we_ref, be_ref,
                g0_ref, g1_ref, g2_ref):
    dist = 1.0 / ea_ref[...]
    dd = dist - cen_ref[...]
    rbf = jnp.exp(-gam_ref[0, 0] * dd * dd)
    t = _silu(jnp.dot(rbf, wr1_ref[...],
                      preferred_element_type=jnp.float32) + br1_ref[...])
    de = jnp.dot(t, wr2_ref[...],
                 preferred_element_type=jnp.float32) + br2_ref[...]
    t2 = _silu(jnp.dot(sh_ref[...], ws1_ref[...],
                       preferred_element_type=jnp.float32) + bs1_ref[...])
    she = jnp.dot(t2, ws2_ref[...],
                  preferred_element_type=jnp.float32) + bs2_ref[...]
    eaf = jnp.concatenate([de, she], axis=1)
    g = _silu(jnp.dot(eaf, we_ref[...],
                      preferred_element_type=jnp.float32) + be_ref[...])
    g0_ref[...] = g[:, 0 * NF:1 * NF]
    g1_ref[...] = g[:, 1 * NF:2 * NF]
    g2_ref[...] = g[:, 2 * NF:3 * NF]


def _gates(sh_p, ea_p, cen, gam, wr1, br1, wr2, br2, ws1, bs1, ws2, bs2,
           we_cat, be_cat):
    full = lambda r, c: pl.BlockSpec((r, c), lambda i: (0, 0))
    return pl.pallas_call(
        _gates_body,
        grid=(E_BLOCKS,),
        in_specs=[
            pl.BlockSpec((BE, SH_L2), lambda i: (i, 0)),
            pl.BlockSpec((BE, 1), lambda i: (i, 0)),
            full(1, RBF_BINS), full(1, 1),
            full(RBF_BINS, 64), full(1, 64),
            full(64, DIST_DIM), full(1, DIST_DIM),
            full(SH_L2, 32), full(1, 32),
            full(32, SH_DIM), full(1, SH_DIM),
            full(NF, 3 * NF), full(1, 3 * NF),
        ],
        out_specs=[pl.BlockSpec((BE, NF), lambda i: (i, 0))] * 3,
        out_shape=[jax.ShapeDtypeStruct((E_PAD, NF), jnp.float32)] * 3,
    )(sh_p, ea_p, cen, gam, wr1, br1, wr2, br2, ws1, bs1, ws2, bs2,
      we_cat, be_cat)


def _sc_body(hm, src_i, dst_i, gate, out0, out1,
             agg, srcv, dstv, gatev, rowsv, msgv, zv, sem):
    c = lax.axis_index("c")
    s = lax.axis_index("s")

    def _zrow(r, _):
        zv[r, pl.ds(0, 16)] = jnp.zeros((16,), jnp.float32)
        zv[r, pl.ds(16, 16)] = jnp.zeros((16,), jnp.float32)
        return 0

    lax.fori_loop(0, ZR, _zrow, 0)

    def _zcp(k, _):
        pltpu.sync_copy(zv, agg.at[pl.ds(s * STRIPE + k * ZR, ZR)])
        return 0

    lax.fori_loop(0, NZ, _zcp, 0)
    plsc.subcore_barrier()

    wid = c * 16 + s
    row_base = wid * NBLK

    def _blk(b, _):
        rb = row_base + b
        pltpu.sync_copy(src_i.at[rb], srcv)
        pltpu.sync_copy(dst_i.at[rb], dstv)
        pltpu.sync_copy(gate.at[rb], gatev)
        pltpu.async_copy(hm.at[srcv], rowsv, sem).wait()

        def _mul(r, _):
            lo = pl.ds(0, 16)
            hi = pl.ds(16, 16)
            msgv[r, lo] = rowsv[r, lo] * gatev[r, lo]
            msgv[r, hi] = rowsv[r, hi] * gatev[r, hi]
            return 0

        lax.fori_loop(0, BLK, _mul, 0)
        pltpu.sync_copy(msgv, agg.at[dstv], add=True)
        return 0

    lax.fori_loop(0, NBLK, _blk, 0)
    plsc.subcore_barrier()

    stripe = pl.ds(s * STRIPE, STRIPE)

    @pl.when(c == 0)
    def _():
        pltpu.sync_copy(agg.at[stripe], out0.at[stripe])

    @pl.when(c == 1)
    def _():
        pltpu.sync_copy(agg.at[stripe], out1.at[stripe])


def _sc_layer(hm, src2d, dst2d, gate3d):
    mesh = plsc.VectorSubcoreMesh(core_axis_name="c", subcore_axis_name="s")
    k = functools.partial(
        pl.kernel,
        mesh=mesh,
        compiler_params=pltpu.CompilerParams(use_tc_tiling_on_sc=False),
        out_type=[
            jax.ShapeDtypeStruct((N_PAD, NF), jnp.float32),
            jax.ShapeDtypeStruct((N_PAD, NF), jnp.float32),
        ],
        scratch_types=[
            pltpu.VMEM_SHARED((N_PAD, NF), jnp.float32),
            pltpu.VMEM((BLK,), jnp.int32),
            pltpu.VMEM((BLK,), jnp.int32),
            pltpu.VMEM((BLK, NF), jnp.float32),
            pltpu.VMEM((BLK, NF), jnp.float32),
            pltpu.VMEM((BLK, NF), jnp.float32),
            pltpu.VMEM((ZR, NF), jnp.float32),
            pltpu.SemaphoreType.DMA,
        ],
    )(_sc_body)
    return k(hm, src2d, dst2d, gate3d)


def _update_body(h_ref, p0_ref, p1_ref, wu_ref, bu_ref, wmn_ref, bmn_ref,
                 hn_ref, hmn_ref):
    agg = p0_ref[...] + p1_ref[...]
    hn = h_ref[...] + _silu(
        jnp.dot(agg, wu_ref[...], preferred_element_type=jnp.float32)
        + bu_ref[...])
    hn_ref[...] = hn
    if hmn_ref is not None:
        hmn_ref[...] = jnp.dot(hn, wmn_ref[...],
                               preferred_element_type=jnp.float32) + bmn_ref[...]


def _update(h, p0, p1, wu, bu, wmn, bmn, last):
    full = lambda r, c: pl.BlockSpec((r, c), lambda i: (0, 0))
    nf_blk = pl.BlockSpec((BN, NF), lambda i: (i, 0))
    nf_out = jax.ShapeDtypeStruct((N_PAD, NF), jnp.float32)
    if last:
        body = lambda h_ref, p0_ref, p1_ref, wu_ref, bu_ref, hn_ref: \
            _update_body(h_ref, p0_ref, p1_ref, wu_ref, bu_ref, None, None,
                         hn_ref, None)
        return pl.pallas_call(
            body,
            grid=(N_BLOCKS,),
            in_specs=[nf_blk, nf_blk, nf_blk, full(NF, NF), full(1, NF)],
            out_specs=nf_blk,
            out_shape=nf_out,
        )(h, p0, p1, wu, bu)
    return pl.pallas_call(
        _update_body,
        grid=(N_BLOCKS,),
        in_specs=[nf_blk, nf_blk, nf_blk, full(NF, NF), full(1, NF),
                  full(NF, NF), full(1, NF)],
        out_specs=[nf_blk, nf_blk],
        out_shape=[nf_out, nf_out],
    )(h, p0, p1, wu, bu, wmn, bmn)


def _pool_body(h_ref, b_ref, wf1_ref, bf1_ref, wf2_ref, bf2_ref, out_ref,
               gsum_ref, cnt_ref):
    i = pl.program_id(0)

    @pl.when(i == 0)
    def _():
        gsum_ref[...] = jnp.zeros_like(gsum_ref)
        cnt_ref[...] = jnp.zeros_like(cnt_ref)

    gids = lax.broadcasted_iota(jnp.int32, (NUM_GRAPHS, BN), 0)
    onehot = (gids == b_ref[...]).astype(jnp.float32)
    gsum_ref[...] += jnp.dot(onehot, h_ref[...],
                             preferred_element_type=jnp.float32)
    cnt_ref[...] += jnp.sum(onehot, axis=1, keepdims=True)

    @pl.when(i == N_BLOCKS - 1)
    def _():
        feats = gsum_ref[...] / jnp.maximum(cnt_ref[...], 1.0)
        z = _silu(jnp.dot(feats, wf1_ref[...],
                          preferred_element_type=jnp.float32) + bf1_ref[...])
        out_ref[...] = jnp.dot(z, wf2_ref[...],
                               preferred_element_type=jnp.float32) + bf2_ref[...]


def _pool_head(h, batch_row, wf1, bf1, wf2, bf2):
    full = lambda r, c: pl.BlockSpec((r, c), lambda i: (0, 0))
    return pl.pallas_call(
        _pool_body,
        grid=(N_BLOCKS,),
        in_specs=[
            pl.BlockSpec((BN, NF), lambda i: (i, 0)),
            pl.BlockSpec((1, BN), lambda i: (0, i)),
            full(NF, 128), full(1, 128), full(128, 1), full(1, 1),
        ],
        out_specs=full(NUM_GRAPHS, 1),
        out_shape=jax.ShapeDtypeStruct((NUM_GRAPHS, 1), jnp.float32),
        scratch_shapes=[
            pltpu.VMEM((NUM_GRAPHS, NF), jnp.float32),
            pltpu.VMEM((NUM_GRAPHS, 1), jnp.float32),
        ],
    )(h, batch_row, wf1, bf1, wf2, bf2)


def kernel(x, edge_index, sh, edge_attr, batch, W_atom, b_atom, W_sh1, b_sh1,
           W_sh2, b_sh2, W_r1, b_r1, W_r2, b_r2, Wm, bm, We, be, Wu, bu,
           W_f1, b_f1, W_f2, b_f2):
    f32 = jnp.float32
    row = lambda v: v.reshape(1, -1)

    npad = E_PAD - E
    pad_ar = jnp.arange(npad, dtype=jnp.int32)
    pad_src = (pad_ar * 17) % N
    pad_dst = N + pad_ar % (N_PAD - N)
    x_p = jnp.pad(x, ((0, N_PAD - N), (0, 0)))
    src_p = jnp.concatenate([edge_index[0], pad_src]).reshape(E_PAD // BLK,
                                                              BLK)
    dst_p = jnp.concatenate([edge_index[1], pad_dst]).reshape(E_PAD // BLK,
                                                              BLK)
    sh_p = jnp.pad(sh, ((0, npad), (0, 0)))
    ea_p = jnp.pad(edge_attr, (0, npad), constant_values=1.0).reshape(E_PAD, 1)
    batch_p = jnp.pad(batch, (0, N_PAD - N),
                      constant_values=NUM_GRAPHS).reshape(1, N_PAD)
    centers = jnp.linspace(0.125, 1.4, RBF_BINS).astype(f32)
    gam = (1.0 / (centers[1] - centers[0])).reshape(1, 1)
    cen = centers.reshape(1, RBF_BINS)
    we_cat = jnp.concatenate([We[0], We[1], We[2]], axis=1)
    be_cat = jnp.concatenate([be[0], be[1], be[2]]).reshape(1, 3 * NF)

    h, hm = _embed(x_p, W_atom, row(b_atom), Wm[0], row(bm[0]))
    g0, g1, g2 = _gates(sh_p, ea_p, cen, gam, W_r1, row(b_r1), W_r2,
                        row(b_r2), W_sh1, row(b_sh1), W_sh2, row(b_sh2),
                        we_cat, be_cat)
    gates = [g.reshape(E_PAD // BLK, BLK, NF) for g in (g0, g1, g2)]

    for i in range(NUM_LAYERS):
        p0, p1 = _sc_layer(hm, src_p, dst_p, gates[i])
        last = i == NUM_LAYERS - 1
        if last:
            h = _update(h, p0, p1, Wu[i], row(bu[i]), None, None, True)
        else:
            h, hm = _update(h, p0, p1, Wu[i], row(bu[i]), Wm[i + 1],
                            row(bm[i + 1]), False)

    out = _pool_head(h, batch_p, W_f1, row(b_f1), W_f2, row(b_f2))
    return out[:, 0]


# speedup vs baseline: 2.9650x; 1.0943x over previous
import functools

import jax
import jax.numpy as jnp
from jax import lax
from jax.experimental import pallas as pl
from jax.experimental.pallas import tpu as pltpu
from jax.experimental.pallas import tpu_sc as plsc

N = 50000
E = 800000
IN_DIM = 92
NF = 32
DIST_DIM = 16
SH_DIM = 16
SH_L2 = 9
NUM_LAYERS = 3
NUM_GRAPHS = 64
RBF_BINS = 64

BN = 1024
N_PAD = 50176
N_BLOCKS = N_PAD // BN
BE = 2048
NW = 32
BLK = 128
NBLK = 196
EPW = BLK * NBLK
E_PAD = NW * EPW
E_BLOCKS = E_PAD // BE
STRIPE = N_PAD // 16
ZR = 196
NZ = STRIPE // ZR


def _silu(v):
    return v * jax.nn.sigmoid(v)


def _embed_body(x_ref, wa_ref, ba_ref, wm0_ref, bm0_ref, h_ref, hm_ref):
    h = jnp.dot(x_ref[...], wa_ref[...],
                preferred_element_type=jnp.float32) + ba_ref[...]
    h_ref[...] = h
    hm_ref[...] = jnp.dot(h, wm0_ref[...],
                          preferred_element_type=jnp.float32) + bm0_ref[...]


def _embed(x_p, wa, ba, wm0, bm0):
    return pl.pallas_call(
        _embed_body,
        grid=(N_BLOCKS,),
        in_specs=[
            pl.BlockSpec((BN, IN_DIM), lambda i: (i, 0)),
            pl.BlockSpec((IN_DIM, NF), lambda i: (0, 0)),
            pl.BlockSpec((1, NF), lambda i: (0, 0)),
            pl.BlockSpec((NF, NF), lambda i: (0, 0)),
            pl.BlockSpec((1, NF), lambda i: (0, 0)),
        ],
        out_specs=[
            pl.BlockSpec((BN, NF), lambda i: (i, 0)),
            pl.BlockSpec((BN, NF), lambda i: (i, 0)),
        ],
        out_shape=[
            jax.ShapeDtypeStruct((N_PAD, NF), jnp.float32),
            jax.ShapeDtypeStruct((N_PAD, NF), jnp.float32),
        ],
    )(x_p, wa, ba, wm0, bm0)


def _gates_body(sh_ref, ea_ref, cen_ref, gam_ref, wr1_ref, br1_ref, wr2_ref,
                br2_ref, ws1_ref, bs1_ref, ws2_ref, bs2_ref, we_ref, be_ref,
                g0_ref, g1_ref, g2_ref):
    dist = 1.0 / ea_ref[...]
    dd = dist - cen_ref[...]
    rbf = jnp.exp(-gam_ref[0, 0] * dd * dd)
    t = _silu(jnp.dot(rbf, wr1_ref[...],
                      preferred_element_type=jnp.float32) + br1_ref[...])
    de = jnp.dot(t, wr2_ref[...],
                 preferred_element_type=jnp.float32) + br2_ref[...]
    t2 = _silu(jnp.dot(sh_ref[...], ws1_ref[...],
                       preferred_element_type=jnp.float32) + bs1_ref[...])
    she = jnp.dot(t2, ws2_ref[...],
                  preferred_element_type=jnp.float32) + bs2_ref[...]
    eaf = jnp.concatenate([de, she], axis=1)
    g = _silu(jnp.dot(eaf, we_ref[...],
                      preferred_element_type=jnp.float32) + be_ref[...])
    q = BE // 4
    for ref, lo in ((g0_ref, 0), (g1_ref, NF), (g2_ref, 2 * NF)):
        ref[...] = jnp.concatenate(
            [g[j * q:(j + 1) * q, lo:lo + NF] for j in range(4)], axis=1)


def _gates(sh_p, ea_p, cen, gam, wr1, br1, wr2, br2, ws1, bs1, ws2, bs2,
           we_cat, be_cat):
    full = lambda r, c: pl.BlockSpec((r, c), lambda i: (0, 0))
    return pl.pallas_call(
        _gates_body,
        grid=(E_BLOCKS,),
        in_specs=[
            pl.BlockSpec((BE, SH_L2), lambda i: (i, 0)),
            pl.BlockSpec((BE, 1), lambda i: (i, 0)),
            full(1, RBF_BINS), full(1, 1),
            full(RBF_BINS, 64), full(1, 64),
            full(64, DIST_DIM), full(1, DIST_DIM),
            full(SH_L2, 32), full(1, 32),
            full(32, SH_DIM), full(1, SH_DIM),
            full(NF, 3 * NF), full(1, 3 * NF),
        ],
        out_specs=[pl.BlockSpec((BE // 4, 128), lambda i: (i, 0))] * 3,
        out_shape=[jax.ShapeDtypeStruct((E_PAD // 4, 128), jnp.float32)] * 3,
    )(sh_p, ea_p, cen, gam, wr1, br1, wr2, br2, ws1, bs1, ws2, bs2,
      we_cat, be_cat)


def _sc_body(hm, src_i, dst_i, gate, out0, out1,
             agg, srcv, dstv, gatev, rowsv, msgv, zv, sem):
    c = lax.axis_index("c")
    s = lax.axis_index("s")

    def _zrow(r, _):
        zv[r, pl.ds(0, 16)] = jnp.zeros((16,), jnp.float32)
        zv[r, pl.ds(16, 16)] = jnp.zeros((16,), jnp.float32)
        return 0

    lax.fori_loop(0, ZR, _zrow, 0)

    def _zcp(k, _):
        pltpu.sync_copy(zv, agg.at[pl.ds(s * STRIPE + k * ZR, ZR)])
        return 0

    lax.fori_loop(0, NZ, _zcp, 0)
    plsc.subcore_barrier()

    wid = c * 16 + s
    row_base = wid * NBLK

    def _blk(b, _):
        rb = row_base + b
        pltpu.sync_copy(src_i.at[rb], srcv)
        pltpu.sync_copy(dst_i.at[rb], dstv)
        pltpu.sync_copy(gate.at[rb], gatev)
        pltpu.async_copy(hm.at[srcv], rowsv, sem).wait()

        def _mul(r, _):
            for q in range(4):
                e = r * 4 + q
                lo = pl.ds(0, 16)
                hi = pl.ds(16, 16)
                msgv[e, lo] = rowsv[e, lo] * gatev[r, pl.ds(q * 32, 16)]
                msgv[e, hi] = rowsv[e, hi] * gatev[r, pl.ds(q * 32 + 16, 16)]
            return 0

        lax.fori_loop(0, BLK // 4, _mul, 0)
        pltpu.sync_copy(msgv, agg.at[dstv], add=True)
        return 0

    lax.fori_loop(0, NBLK, _blk, 0)
    plsc.subcore_barrier()

    stripe = pl.ds(s * STRIPE, STRIPE)

    @pl.when(c == 0)
    def _():
        pltpu.sync_copy(agg.at[stripe], out0.at[stripe])

    @pl.when(c == 1)
    def _():
        pltpu.sync_copy(agg.at[stripe], out1.at[stripe])


def _sc_layer(hm, src2d, dst2d, gate3d):
    mesh = plsc.VectorSubcoreMesh(core_axis_name="c", subcore_axis_name="s")
    k = functools.partial(
        pl.kernel,
        mesh=mesh,
        compiler_params=pltpu.CompilerParams(use_tc_tiling_on_sc=False),
        out_type=[
            jax.ShapeDtypeStruct((N_PAD, NF), jnp.float32),
            jax.ShapeDtypeStruct((N_PAD, NF), jnp.float32),
        ],
        scratch_types=[
            pltpu.VMEM_SHARED((N_PAD, NF), jnp.float32),
            pltpu.VMEM((BLK,), jnp.int32),
            pltpu.VMEM((BLK,), jnp.int32),
            pltpu.VMEM((BLK // 4, 128), jnp.float32),
            pltpu.VMEM((BLK, NF), jnp.float32),
            pltpu.VMEM((BLK, NF), jnp.float32),
            pltpu.VMEM((ZR, NF), jnp.float32),
            pltpu.SemaphoreType.DMA,
        ],
    )(_sc_body)
    return k(hm, src2d, dst2d, gate3d)


def _update_body(h_ref, p0_ref, p1_ref, wu_ref, bu_ref, wmn_ref, bmn_ref,
                 hn_ref, hmn_ref):
    agg = p0_ref[...] + p1_ref[...]
    hn = h_ref[...] + _silu(
        jnp.dot(agg, wu_ref[...], preferred_element_type=jnp.float32)
        + bu_ref[...])
    hn_ref[...] = hn
    if hmn_ref is not None:
        hmn_ref[...] = jnp.dot(hn, wmn_ref[...],
                               preferred_element_type=jnp.float32) + bmn_ref[...]


def _update(h, p0, p1, wu, bu, wmn, bmn, last):
    full = lambda r, c: pl.BlockSpec((r, c), lambda i: (0, 0))
    nf_blk = pl.BlockSpec((BN, NF), lambda i: (i, 0))
    nf_out = jax.ShapeDtypeStruct((N_PAD, NF), jnp.float32)
    if last:
        body = lambda h_ref, p0_ref, p1_ref, wu_ref, bu_ref, hn_ref: \
            _update_body(h_ref, p0_ref, p1_ref, wu_ref, bu_ref, None, None,
                         hn_ref, None)
        return pl.pallas_call(
            body,
            grid=(N_BLOCKS,),
            in_specs=[nf_blk, nf_blk, nf_blk, full(NF, NF), full(1, NF)],
            out_specs=nf_blk,
            out_shape=nf_out,
        )(h, p0, p1, wu, bu)
    return pl.pallas_call(
        _update_body,
        grid=(N_BLOCKS,),
        in_specs=[nf_blk, nf_blk, nf_blk, full(NF, NF), full(1, NF),
                  full(NF, NF), full(1, NF)],
        out_specs=[nf_blk, nf_blk],
        out_shape=[nf_out, nf_out],
    )(h, p0, p1, wu, bu, wmn, bmn)


def _pool_body(h_ref, b_ref, wf1_ref, bf1_ref, wf2_ref, bf2_ref, out_ref,
               gsum_ref, cnt_ref):
    i = pl.program_id(0)

    @pl.when(i == 0)
    def _():
        gsum_ref[...] = jnp.zeros_like(gsum_ref)
        cnt_ref[...] = jnp.zeros_like(cnt_ref)

    gids = lax.broadcasted_iota(jnp.int32, (NUM_GRAPHS, BN), 0)
    onehot = (gids == b_ref[...]).astype(jnp.float32)
    gsum_ref[...] += jnp.dot(onehot, h_ref[...],
                             preferred_element_type=jnp.float32)
    cnt_ref[...] += jnp.sum(onehot, axis=1, keepdims=True)

    @pl.when(i == N_BLOCKS - 1)
    def _():
        feats = gsum_ref[...] / jnp.maximum(cnt_ref[...], 1.0)
        z = _silu(jnp.dot(feats, wf1_ref[...],
                          preferred_element_type=jnp.float32) + bf1_ref[...])
        out_ref[...] = jnp.dot(z, wf2_ref[...],
                               preferred_element_type=jnp.float32) + bf2_ref[...]


def _pool_head(h, batch_row, wf1, bf1, wf2, bf2):
    full = lambda r, c: pl.BlockSpec((r, c), lambda i: (0, 0))
    return pl.pallas_call(
        _pool_body,
        grid=(N_BLOCKS,),
        in_specs=[
            pl.BlockSpec((BN, NF), lambda i: (i, 0)),
            pl.BlockSpec((1, BN), lambda i: (0, i)),
            full(NF, 128), full(1, 128), full(128, 1), full(1, 1),
        ],
        out_specs=full(NUM_GRAPHS, 1),
        out_shape=jax.ShapeDtypeStruct((NUM_GRAPHS, 1), jnp.float32),
        scratch_shapes=[
            pltpu.VMEM((NUM_GRAPHS, NF), jnp.float32),
            pltpu.VMEM((NUM_GRAPHS, 1), jnp.float32),
        ],
    )(h, batch_row, wf1, bf1, wf2, bf2)


def kernel(x, edge_index, sh, edge_attr, batch, W_atom, b_atom, W_sh1, b_sh1,
           W_sh2, b_sh2, W_r1, b_r1, W_r2, b_r2, Wm, bm, We, be, Wu, bu,
           W_f1, b_f1, W_f2, b_f2):
    f32 = jnp.float32
    row = lambda v: v.reshape(1, -1)

    npad = E_PAD - E
    pad_ar = jnp.arange(npad, dtype=jnp.int32)
    pad_src = (pad_ar * 17) % N
    pad_dst = N + pad_ar % (N_PAD - N)
    x_p = jnp.pad(x, ((0, N_PAD - N), (0, 0)))
    repack = lambda v: v.reshape(E_BLOCKS, 4, BE // 4).transpose(
        0, 2, 1).reshape(E_PAD // BLK, BLK)
    src_p = repack(jnp.concatenate([edge_index[0], pad_src]))
    dst_p = repack(jnp.concatenate([edge_index[1], pad_dst]))
    sh_p = jnp.pad(sh, ((0, npad), (0, 0)))
    ea_p = jnp.pad(edge_attr, (0, npad), constant_values=1.0).reshape(E_PAD, 1)
    batch_p = jnp.pad(batch, (0, N_PAD - N),
                      constant_values=NUM_GRAPHS).reshape(1, N_PAD)
    centers = jnp.linspace(0.125, 1.4, RBF_BINS).astype(f32)
    gam = (1.0 / (centers[1] - centers[0])).reshape(1, 1)
    cen = centers.reshape(1, RBF_BINS)
    we_cat = jnp.concatenate([We[0], We[1], We[2]], axis=1)
    be_cat = jnp.concatenate([be[0], be[1], be[2]]).reshape(1, 3 * NF)

    h, hm = _embed(x_p, W_atom, row(b_atom), Wm[0], row(bm[0]))
    g0, g1, g2 = _gates(sh_p, ea_p, cen, gam, W_r1, row(b_r1), W_r2,
                        row(b_r2), W_sh1, row(b_sh1), W_sh2, row(b_sh2),
                        we_cat, be_cat)
    gates = [g.reshape(E_PAD // BLK, BLK // 4, 128) for g in (g0, g1, g2)]

    for i in range(NUM_LAYERS):
        p0, p1 = _sc_layer(hm, src_p, dst_p, gates[i])
        last = i == NUM_LAYERS - 1
        if last:
            h = _update(h, p0, p1, Wu[i], row(bu[i]), None, None, True)
        else:
            h, hm = _update(h, p0, p1, Wu[i], row(bu[i]), Wm[i + 1],
                            row(bm[i + 1]), False)

    out = _pool_head(h, batch_p, W_f1, row(b_f1), W_f2, row(b_f2))
    return out[:, 0]


# speedup vs baseline: 4.0215x; 1.3564x over previous
import functools

import jax
import jax.numpy as jnp
from jax import lax
from jax.experimental import pallas as pl
from jax.experimental.pallas import tpu as pltpu
from jax.experimental.pallas import tpu_sc as plsc

N = 50000
E = 800000
IN_DIM = 92
NF = 32
DIST_DIM = 16
SH_DIM = 16
SH_L2 = 9
NUM_LAYERS = 3
NUM_GRAPHS = 64
RBF_BINS = 64

BN = 1024
N_PAD = 50176
N_BLOCKS = N_PAD // BN
BE = 2048
NW = 32
BLK = 128
NBLK = 196
EPW = BLK * NBLK
E_PAD = NW * EPW
E_BLOCKS = E_PAD // BE
STRIPE = N_PAD // 16
ZR = 196
NZ = STRIPE // ZR


def _silu(v):
    return v * jax.nn.sigmoid(v)


def _embed_body(x_ref, wa_ref, ba_ref, wm0_ref, bm0_ref, h_ref, hm_ref):
    h = jnp.dot(x_ref[...], wa_ref[...],
                preferred_element_type=jnp.float32) + ba_ref[...]
    h_ref[...] = h
    hm_ref[...] = jnp.dot(h, wm0_ref[...],
                          preferred_element_type=jnp.float32) + bm0_ref[...]


def _embed(x_p, wa, ba, wm0, bm0):
    return pl.pallas_call(
        _embed_body,
        grid=(N_BLOCKS,),
        in_specs=[
            pl.BlockSpec((BN, IN_DIM), lambda i: (i, 0)),
            pl.BlockSpec((IN_DIM, NF), lambda i: (0, 0)),
            pl.BlockSpec((1, NF), lambda i: (0, 0)),
            pl.BlockSpec((NF, NF), lambda i: (0, 0)),
            pl.BlockSpec((1, NF), lambda i: (0, 0)),
        ],
        out_specs=[
            pl.BlockSpec((BN, NF), lambda i: (i, 0)),
            pl.BlockSpec((BN, NF), lambda i: (i, 0)),
        ],
        out_shape=[
            jax.ShapeDtypeStruct((N_PAD, NF), jnp.float32),
            jax.ShapeDtypeStruct((N_PAD, NF), jnp.float32),
        ],
    )(x_p, wa, ba, wm0, bm0)


def _gates_body(sh_ref, ea_ref, cen_ref, gam_ref, wr1_ref, br1_ref, wr2_ref,
                br2_ref, ws1_ref, bs1_ref, ws2_ref, bs2_ref, we_ref, be_ref,
                g0_ref, g1_ref, g2_ref):
    dist = 1.0 / ea_ref[...]
    dd = dist - cen_ref[...]
    rbf = jnp.exp(-gam_ref[0, 0] * dd * dd)
    t = _silu(jnp.dot(rbf, wr1_ref[...],
                      preferred_element_type=jnp.float32) + br1_ref[...])
    de = jnp.dot(t, wr2_ref[...],
                 preferred_element_type=jnp.float32) + br2_ref[...]
    t2 = _silu(jnp.dot(sh_ref[...], ws1_ref[...],
                       preferred_element_type=jnp.float32) + bs1_ref[...])
    she = jnp.dot(t2, ws2_ref[...],
                  preferred_element_type=jnp.float32) + bs2_ref[...]
    eaf = jnp.concatenate([de, she], axis=1)
    g = _silu(jnp.dot(eaf, we_ref[...],
                      preferred_element_type=jnp.float32) + be_ref[...])
    q = BE // 4
    for ref, lo in ((g0_ref, 0), (g1_ref, NF), (g2_ref, 2 * NF)):
        ref[...] = jnp.concatenate(
            [g[j * q:(j + 1) * q, lo:lo + NF] for j in range(4)], axis=1)


def _gates(sh_p, ea_p, cen, gam, wr1, br1, wr2, br2, ws1, bs1, ws2, bs2,
           we_cat, be_cat):
    full = lambda r, c: pl.BlockSpec((r, c), lambda i: (0, 0))
    return pl.pallas_call(
        _gates_body,
        grid=(E_BLOCKS,),
        in_specs=[
            pl.BlockSpec((BE, SH_L2), lambda i: (i, 0)),
            pl.BlockSpec((BE, 1), lambda i: (i, 0)),
            full(1, RBF_BINS), full(1, 1),
            full(RBF_BINS, 64), full(1, 64),
            full(64, DIST_DIM), full(1, DIST_DIM),
            full(SH_L2, 32), full(1, 32),
            full(32, SH_DIM), full(1, SH_DIM),
            full(NF, 3 * NF), full(1, 3 * NF),
        ],
        out_specs=[pl.BlockSpec((BE // 4, 128), lambda i: (i, 0))] * 3,
        out_shape=[jax.ShapeDtypeStruct((E_PAD // 4, 128), jnp.float32)] * 3,
    )(sh_p, ea_p, cen, gam, wr1, br1, wr2, br2, ws1, bs1, ws2, bs2,
      we_cat, be_cat)


def _sc_body(hm, src_i, dst_i, gate, out0, out1, agg,
             srcv0, srcv1, dstv0, dstv1, gatev0, gatev1, rowsv0, rowsv1,
             msgv, zv, sem_i0, sem_i1, sem_g0, sem_g1):
    c = lax.axis_index("c")
    s = lax.axis_index("s")
    srcv = (srcv0, srcv1)
    dstv = (dstv0, dstv1)
    gatev = (gatev0, gatev1)
    rowsv = (rowsv0, rowsv1)
    sem_i = (sem_i0, sem_i1)
    sem_g = (sem_g0, sem_g1)

    wid = c * 16 + s
    row_base = wid * NBLK

    def start_in(b, p):
        pltpu.async_copy(src_i.at[row_base + b], srcv[p], sem_i[p])
        pltpu.async_copy(dst_i.at[row_base + b], dstv[p], sem_i[p])
        pltpu.async_copy(gate.at[row_base + b], gatev[p], sem_i[p])

    def wait_in(p):
        pltpu.make_async_copy(src_i.at[row_base], srcv[p], sem_i[p]).wait()
        pltpu.make_async_copy(dst_i.at[row_base], dstv[p], sem_i[p]).wait()
        pltpu.make_async_copy(gate.at[row_base], gatev[p], sem_i[p]).wait()

    def start_gather(p):
        pltpu.async_copy(hm.at[srcv[p]], rowsv[p], sem_g[p])

    def wait_gather(p):
        pltpu.make_async_copy(hm.at[srcv[p]], rowsv[p], sem_g[p]).wait()

    start_in(0, 0)

    def _zrow(r, _):
        zv[r, pl.ds(0, 16)] = jnp.zeros((16,), jnp.float32)
        zv[r, pl.ds(16, 16)] = jnp.zeros((16,), jnp.float32)
        return 0

    lax.fori_loop(0, ZR, _zrow, 0)

    def _zcp(k, _):
        pltpu.sync_copy(zv, agg.at[pl.ds(s * STRIPE + k * ZR, ZR)])
        return 0

    lax.fori_loop(0, NZ, _zcp, 0)

    wait_in(0)
    start_gather(0)
    start_in(1, 1)
    plsc.subcore_barrier()

    def _mul(p):
        rv = rowsv[p]
        gv = gatev[p]

        def body(r, _):
            for q in range(4):
                e = r * 4 + q
                lo = pl.ds(0, 16)
                hi = pl.ds(16, 16)
                msgv[e, lo] = rv[e, lo] * gv[r, pl.ds(q * 32, 16)]
                msgv[e, hi] = rv[e, hi] * gv[r, pl.ds(q * 32 + 16, 16)]
            return 0

        lax.fori_loop(0, BLK // 4, body, 0)

    def _pair(i, _):
        for p in (0, 1):
            g = 2 * i + p
            q = 1 - p
            wait_gather(p)

            @pl.when(g + 1 < NBLK)
            def _():
                wait_in(q)
                start_gather(q)

            _mul(p)
            pltpu.sync_copy(msgv, agg.at[dstv[p]], add=True)

            @pl.when(g + 2 < NBLK)
            def _():
                start_in(g + 2, p)

        return 0

    lax.fori_loop(0, NBLK // 2, _pair, 0)
    plsc.subcore_barrier()

    stripe = pl.ds(s * STRIPE, STRIPE)

    @pl.when(c == 0)
    def _():
        pltpu.sync_copy(agg.at[stripe], out0.at[stripe])

    @pl.when(c == 1)
    def _():
        pltpu.sync_copy(agg.at[stripe], out1.at[stripe])


def _sc_layer(hm, src2d, dst2d, gate3d):
    mesh = plsc.VectorSubcoreMesh(core_axis_name="c", subcore_axis_name="s")
    k = functools.partial(
        pl.kernel,
        mesh=mesh,
        compiler_params=pltpu.CompilerParams(use_tc_tiling_on_sc=False),
        out_type=[
            jax.ShapeDtypeStruct((N_PAD, NF), jnp.float32),
            jax.ShapeDtypeStruct((N_PAD, NF), jnp.float32),
        ],
        scratch_types=[
            pltpu.VMEM_SHARED((N_PAD, NF), jnp.float32),
            pltpu.VMEM((BLK,), jnp.int32),
            pltpu.VMEM((BLK,), jnp.int32),
            pltpu.VMEM((BLK,), jnp.int32),
            pltpu.VMEM((BLK,), jnp.int32),
            pltpu.VMEM((BLK // 4, 128), jnp.float32),
            pltpu.VMEM((BLK // 4, 128), jnp.float32),
            pltpu.VMEM((BLK, NF), jnp.float32),
            pltpu.VMEM((BLK, NF), jnp.float32),
            pltpu.VMEM((BLK, NF), jnp.float32),
            pltpu.VMEM((ZR, NF), jnp.float32),
            pltpu.SemaphoreType.DMA,
            pltpu.SemaphoreType.DMA,
            pltpu.SemaphoreType.DMA,
            pltpu.SemaphoreType.DMA,
        ],
    )(_sc_body)
    return k(hm, src2d, dst2d, gate3d)


def _update_body(h_ref, p0_ref, p1_ref, wu_ref, bu_ref, wmn_ref, bmn_ref,
                 hn_ref, hmn_ref):
    agg = p0_ref[...] + p1_ref[...]
    hn = h_ref[...] + _silu(
        jnp.dot(agg, wu_ref[...], preferred_element_type=jnp.float32)
        + bu_ref[...])
    hn_ref[...] = hn
    if hmn_ref is not None:
        hmn_ref[...] = jnp.dot(hn, wmn_ref[...],
                               preferred_element_type=jnp.float32) + bmn_ref[...]


def _update(h, p0, p1, wu, bu, wmn, bmn, last):
    full = lambda r, c: pl.BlockSpec((r, c), lambda i: (0, 0))
    nf_blk = pl.BlockSpec((BN, NF), lambda i: (i, 0))
    nf_out = jax.ShapeDtypeStruct((N_PAD, NF), jnp.float32)
    if last:
        body = lambda h_ref, p0_ref, p1_ref, wu_ref, bu_ref, hn_ref: \
            _update_body(h_ref, p0_ref, p1_ref, wu_ref, bu_ref, None, None,
                         hn_ref, None)
        return pl.pallas_call(
            body,
            grid=(N_BLOCKS,),
            in_specs=[nf_blk, nf_blk, nf_blk, full(NF, NF), full(1, NF)],
            out_specs=nf_blk,
            out_shape=nf_out,
        )(h, p0, p1, wu, bu)
    return pl.pallas_call(
        _update_body,
        grid=(N_BLOCKS,),
        in_specs=[nf_blk, nf_blk, nf_blk, full(NF, NF), full(1, NF),
                  full(NF, NF), full(1, NF)],
        out_specs=[nf_blk, nf_blk],
        out_shape=[nf_out, nf_out],
    )(h, p0, p1, wu, bu, wmn, bmn)


def _pool_body(h_ref, b_ref, wf1_ref, bf1_ref, wf2_ref, bf2_ref, out_ref,
               gsum_ref, cnt_ref):
    i = pl.program_id(0)

    @pl.when(i == 0)
    def _():
        gsum_ref[...] = jnp.zeros_like(gsum_ref)
        cnt_ref[...] = jnp.zeros_like(cnt_ref)

    gids = lax.broadcasted_iota(jnp.int32, (NUM_GRAPHS, BN), 0)
    onehot = (gids == b_ref[...]).astype(jnp.float32)
    gsum_ref[...] += jnp.dot(onehot, h_ref[...],
                             preferred_element_type=jnp.float32)
    cnt_ref[...] += jnp.sum(onehot, axis=1, keepdims=True)

    @pl.when(i == N_BLOCKS - 1)
    def _():
        feats = gsum_ref[...] / jnp.maximum(cnt_ref[...], 1.0)
        z = _silu(jnp.dot(feats, wf1_ref[...],
                          preferred_element_type=jnp.float32) + bf1_ref[...])
        out_ref[...] = jnp.dot(z, wf2_ref[...],
                               preferred_element_type=jnp.float32) + bf2_ref[...]


def _pool_head(h, batch_row, wf1, bf1, wf2, bf2):
    full = lambda r, c: pl.BlockSpec((r, c), lambda i: (0, 0))
    return pl.pallas_call(
        _pool_body,
        grid=(N_BLOCKS,),
        in_specs=[
            pl.BlockSpec((BN, NF), lambda i: (i, 0)),
            pl.BlockSpec((1, BN), lambda i: (0, i)),
            full(NF, 128), full(1, 128), full(128, 1), full(1, 1),
        ],
        out_specs=full(NUM_GRAPHS, 1),
        out_shape=jax.ShapeDtypeStruct((NUM_GRAPHS, 1), jnp.float32),
        scratch_shapes=[
            pltpu.VMEM((NUM_GRAPHS, NF), jnp.float32),
            pltpu.VMEM((NUM_GRAPHS, 1), jnp.float32),
        ],
    )(h, batch_row, wf1, bf1, wf2, bf2)


def kernel(x, edge_index, sh, edge_attr, batch, W_atom, b_atom, W_sh1, b_sh1,
           W_sh2, b_sh2, W_r1, b_r1, W_r2, b_r2, Wm, bm, We, be, Wu, bu,
           W_f1, b_f1, W_f2, b_f2):
    f32 = jnp.float32
    row = lambda v: v.reshape(1, -1)

    npad = E_PAD - E
    pad_ar = jnp.arange(npad, dtype=jnp.int32)
    pad_src = (pad_ar * 17) % N
    pad_dst = N + pad_ar % (N_PAD - N)
    x_p = jnp.pad(x, ((0, N_PAD - N), (0, 0)))
    repack = lambda v: v.reshape(E_BLOCKS, 4, BE // 4).transpose(
        0, 2, 1).reshape(E_PAD // BLK, BLK)
    src_p = repack(jnp.concatenate([edge_index[0], pad_src]))
    dst_p = repack(jnp.concatenate([edge_index[1], pad_dst]))
    sh_p = jnp.pad(sh, ((0, npad), (0, 0)))
    ea_p = jnp.pad(edge_attr, (0, npad), constant_values=1.0).reshape(E_PAD, 1)
    batch_p = jnp.pad(batch, (0, N_PAD - N),
                      constant_values=NUM_GRAPHS).reshape(1, N_PAD)
    centers = jnp.linspace(0.125, 1.4, RBF_BINS).astype(f32)
    gam = (1.0 / (centers[1] - centers[0])).reshape(1, 1)
    cen = centers.reshape(1, RBF_BINS)
    we_cat = jnp.concatenate([We[0], We[1], We[2]], axis=1)
    be_cat = jnp.concatenate([be[0], be[1], be[2]]).reshape(1, 3 * NF)

    h, hm = _embed(x_p, W_atom, row(b_atom), Wm[0], row(bm[0]))
    g0, g1, g2 = _gates(sh_p, ea_p, cen, gam, W_r1, row(b_r1), W_r2,
                        row(b_r2), W_sh1, row(b_sh1), W_sh2, row(b_sh2),
                        we_cat, be_cat)
    gates = [g.reshape(E_PAD // BLK, BLK // 4, 128) for g in (g0, g1, g2)]

    for i in range(NUM_LAYERS):
        p0, p1 = _sc_layer(hm, src_p, dst_p, gates[i])
        last = i == NUM_LAYERS - 1
        if last:
            h = _update(h, p0, p1, Wu[i], row(bu[i]), None, None, True)
        else:
            h, hm = _update(h, p0, p1, Wu[i], row(bu[i]), Wm[i + 1],
                            row(bm[i + 1]), False)

    out = _pool_head(h, batch_p, W_f1, row(b_f1), W_f2, row(b_f2))
    return out[:, 0]


# speedup vs baseline: 4.0229x; 1.0004x over previous
import functools

import jax
import jax.numpy as jnp
from jax import lax
from jax.experimental import pallas as pl
from jax.experimental.pallas import tpu as pltpu
from jax.experimental.pallas import tpu_sc as plsc

N = 50000
E = 800000
IN_DIM = 92
NF = 32
DIST_DIM = 16
SH_DIM = 16
SH_L2 = 9
NUM_LAYERS = 3
NUM_GRAPHS = 64
RBF_BINS = 64

BN = 1024
N_PAD = 50176
N_BLOCKS = N_PAD // BN
BE = 2048
NW = 32
BLK = 128
NBLK = 196
EPW = BLK * NBLK
E_PAD = NW * EPW
E_BLOCKS = E_PAD // BE
STRIPE = N_PAD // 16
ZR = 196
NZ = STRIPE // ZR


def _silu(v):
    return v * jax.nn.sigmoid(v)


def _embed_body(x_ref, wa_ref, ba_ref, wm0_ref, bm0_ref, h_ref, hm_ref):
    h = jnp.dot(x_ref[...], wa_ref[...],
                preferred_element_type=jnp.float32) + ba_ref[...]
    h_ref[...] = h
    hm_ref[...] = jnp.dot(h, wm0_ref[...],
                          preferred_element_type=jnp.float32) + bm0_ref[...]


def _embed(x_p, wa, ba, wm0, bm0):
    return pl.pallas_call(
        _embed_body,
        grid=(N_BLOCKS,),
        in_specs=[
            pl.BlockSpec((BN, IN_DIM), lambda i: (i, 0)),
            pl.BlockSpec((IN_DIM, NF), lambda i: (0, 0)),
            pl.BlockSpec((1, NF), lambda i: (0, 0)),
            pl.BlockSpec((NF, NF), lambda i: (0, 0)),
            pl.BlockSpec((1, NF), lambda i: (0, 0)),
        ],
        out_specs=[
            pl.BlockSpec((BN, NF), lambda i: (i, 0)),
            pl.BlockSpec((BN, NF), lambda i: (i, 0)),
        ],
        out_shape=[
            jax.ShapeDtypeStruct((N_PAD, NF), jnp.float32),
            jax.ShapeDtypeStruct((N_PAD, NF), jnp.float32),
        ],
    )(x_p, wa, ba, wm0, bm0)


def _gates_body(sh_ref, ea_ref, cen_ref, gam_ref, wr1_ref, br1_ref, wr2_ref,
                br2_ref, ws1_ref, bs1_ref, ws2_ref, bs2_ref, we_ref, be_ref,
                g0_ref, g1_ref, g2_ref):
    dist = 1.0 / ea_ref[...]
    dd = dist - cen_ref[...]
    rbf = jnp.exp(-gam_ref[0, 0] * dd * dd)
    t = _silu(jnp.dot(rbf, wr1_ref[...],
                      preferred_element_type=jnp.float32) + br1_ref[...])
    de = jnp.dot(t, wr2_ref[...],
                 preferred_element_type=jnp.float32) + br2_ref[...]
    t2 = _silu(jnp.dot(sh_ref[...], ws1_ref[...],
                       preferred_element_type=jnp.float32) + bs1_ref[...])
    she = jnp.dot(t2, ws2_ref[...],
                  preferred_element_type=jnp.float32) + bs2_ref[...]
    eaf = jnp.concatenate([de, she], axis=1)
    g = _silu(jnp.dot(eaf, we_ref[...],
                      preferred_element_type=jnp.float32) + be_ref[...])
    q = BE // 4
    for ref, lo in ((g0_ref, 0), (g1_ref, NF), (g2_ref, 2 * NF)):
        ref[...] = jnp.concatenate(
            [g[j * q:(j + 1) * q, lo:lo + NF] for j in range(4)], axis=1)


def _gates(sh_p, ea_p, cen, gam, wr1, br1, wr2, br2, ws1, bs1, ws2, bs2,
           we_cat, be_cat):
    full = lambda r, c: pl.BlockSpec((r, c), lambda i: (0, 0))
    return pl.pallas_call(
        _gates_body,
        grid=(E_BLOCKS,),
        in_specs=[
            pl.BlockSpec((BE, SH_L2), lambda i: (i, 0)),
            pl.BlockSpec((BE, 1), lambda i: (i, 0)),
            full(1, RBF_BINS), full(1, 1),
            full(RBF_BINS, 64), full(1, 64),
            full(64, DIST_DIM), full(1, DIST_DIM),
            full(SH_L2, 32), full(1, 32),
            full(32, SH_DIM), full(1, SH_DIM),
            full(NF, 3 * NF), full(1, 3 * NF),
        ],
        out_specs=[pl.BlockSpec((BE // 4, 128), lambda i: (i, 0))] * 3,
        out_shape=[jax.ShapeDtypeStruct((E_PAD // 4, 128), jnp.float32)] * 3,
    )(sh_p, ea_p, cen, gam, wr1, br1, wr2, br2, ws1, bs1, ws2, bs2,
      we_cat, be_cat)


def _sc_body(hm, src_i, dst_i, gate, out0, out1, agg,
             srcv0, srcv1, dstv0, dstv1, gatev0, gatev1, rowsv0, rowsv1,
             msgv, zv, sem_i0, sem_i1, sem_g0, sem_g1):
    c = lax.axis_index("c")
    s = lax.axis_index("s")
    srcv = (srcv0, srcv1)
    dstv = (dstv0, dstv1)
    gatev = (gatev0, gatev1)
    rowsv = (rowsv0, rowsv1)
    sem_i = (sem_i0, sem_i1)
    sem_g = (sem_g0, sem_g1)

    wid = c * 16 + s
    row_base = wid * NBLK

    def start_in(b, p):
        pltpu.async_copy(src_i.at[row_base + b], srcv[p], sem_i[p])
        pltpu.async_copy(dst_i.at[row_base + b], dstv[p], sem_i[p])
        pltpu.async_copy(gate.at[row_base + b], gatev[p], sem_i[p])

    def wait_in(p):
        pltpu.make_async_copy(src_i.at[row_base], srcv[p], sem_i[p]).wait()
        pltpu.make_async_copy(dst_i.at[row_base], dstv[p], sem_i[p]).wait()
        pltpu.make_async_copy(gate.at[row_base], gatev[p], sem_i[p]).wait()

    def start_gather(p):
        pltpu.async_copy(hm.at[srcv[p]], rowsv[p], sem_g[p])

    def wait_gather(p):
        pltpu.make_async_copy(hm.at[srcv[p]], rowsv[p], sem_g[p]).wait()

    start_in(0, 0)

    def _zrow(r, _):
        zv[r, pl.ds(0, 16)] = jnp.zeros((16,), jnp.float32)
        zv[r, pl.ds(16, 16)] = jnp.zeros((16,), jnp.float32)
        return 0

    lax.fori_loop(0, ZR, _zrow, 0)

    def _zcp(k, _):
        pltpu.sync_copy(zv, agg.at[pl.ds(s * STRIPE + k * ZR, ZR)])
        return 0

    lax.fori_loop(0, NZ, _zcp, 0)

    wait_in(0)
    start_gather(0)
    start_in(1, 1)
    plsc.subcore_barrier()

    def _mul(p):
        rv = rowsv[p]
        gv = gatev[p]

        def body(r2, _):
            for u in range(2):
                r = r2 * 2 + u
                for q in range(4):
                    e = r * 4 + q
                    lo = pl.ds(0, 16)
                    hi = pl.ds(16, 16)
                    msgv[e, lo] = rv[e, lo] * gv[r, pl.ds(q * 32, 16)]
                    msgv[e, hi] = rv[e, hi] * gv[r, pl.ds(q * 32 + 16, 16)]
            return 0

        lax.fori_loop(0, BLK // 8, body, 0)

    def _pair(i, _):
        for p in (0, 1):
            g = 2 * i + p
            q = 1 - p
            wait_gather(p)

            @pl.when(g + 1 < NBLK)
            def _():
                wait_in(q)
                start_gather(q)

            _mul(p)
            pltpu.sync_copy(msgv, agg.at[dstv[p]], add=True)

            @pl.when(g + 2 < NBLK)
            def _():
                start_in(g + 2, p)

        return 0

    lax.fori_loop(0, NBLK // 2, _pair, 0)
    plsc.subcore_barrier()

    stripe = pl.ds(s * STRIPE, STRIPE)

    @pl.when(c == 0)
    def _():
        pltpu.sync_copy(agg.at[stripe], out0.at[stripe])

    @pl.when(c == 1)
    def _():
        pltpu.sync_copy(agg.at[stripe], out1.at[stripe])


def _sc_layer(hm, src2d, dst2d, gate3d):
    mesh = plsc.VectorSubcoreMesh(core_axis_name="c", subcore_axis_name="s")
    k = functools.partial(
        pl.kernel,
        mesh=mesh,
        compiler_params=pltpu.CompilerParams(use_tc_tiling_on_sc=False),
        out_type=[
            jax.ShapeDtypeStruct((N_PAD, NF), jnp.float32),
            jax.ShapeDtypeStruct((N_PAD, NF), jnp.float32),
        ],
        scratch_types=[
            pltpu.VMEM_SHARED((N_PAD, NF), jnp.float32),
            pltpu.VMEM((BLK,), jnp.int32),
            pltpu.VMEM((BLK,), jnp.int32),
            pltpu.VMEM((BLK,), jnp.int32),
            pltpu.VMEM((BLK,), jnp.int32),
            pltpu.VMEM((BLK // 4, 128), jnp.float32),
            pltpu.VMEM((BLK // 4, 128), jnp.float32),
            pltpu.VMEM((BLK, NF), jnp.float32),
            pltpu.VMEM((BLK, NF), jnp.float32),
            pltpu.VMEM((BLK, NF), jnp.float32),
            pltpu.VMEM((ZR, NF), jnp.float32),
            pltpu.SemaphoreType.DMA,
            pltpu.SemaphoreType.DMA,
            pltpu.SemaphoreType.DMA,
            pltpu.SemaphoreType.DMA,
        ],
    )(_sc_body)
    return k(hm, src2d, dst2d, gate3d)


def _update_body(h_ref, p0_ref, p1_ref, wu_ref, bu_ref, wmn_ref, bmn_ref,
                 hn_ref, hmn_ref):
    agg = p0_ref[...] + p1_ref[...]
    hn = h_ref[...] + _silu(
        jnp.dot(agg, wu_ref[...], preferred_element_type=jnp.float32)
        + bu_ref[...])
    hn_ref[...] = hn
    if hmn_ref is not None:
        hmn_ref[...] = jnp.dot(hn, wmn_ref[...],
                               preferred_element_type=jnp.float32) + bmn_ref[...]


def _update(h, p0, p1, wu, bu, wmn, bmn, last):
    full = lambda r, c: pl.BlockSpec((r, c), lambda i: (0, 0))
    nf_blk = pl.BlockSpec((BN, NF), lambda i: (i, 0))
    nf_out = jax.ShapeDtypeStruct((N_PAD, NF), jnp.float32)
    if last:
        body = lambda h_ref, p0_ref, p1_ref, wu_ref, bu_ref, hn_ref: \
            _update_body(h_ref, p0_ref, p1_ref, wu_ref, bu_ref, None, None,
                         hn_ref, None)
        return pl.pallas_call(
            body,
            grid=(N_BLOCKS,),
            in_specs=[nf_blk, nf_blk, nf_blk, full(NF, NF), full(1, NF)],
            out_specs=nf_blk,
            out_shape=nf_out,
        )(h, p0, p1, wu, bu)
    return pl.pallas_call(
        _update_body,
        grid=(N_BLOCKS,),
        in_specs=[nf_blk, nf_blk, nf_blk, full(NF, NF), full(1, NF),
                  full(NF, NF), full(1, NF)],
        out_specs=[nf_blk, nf_blk],
        out_shape=[nf_out, nf_out],
    )(h, p0, p1, wu, bu, wmn, bmn)


def _pool_body(h_ref, b_ref, wf1_ref, bf1_ref, wf2_ref, bf2_ref, out_ref,
               gsum_ref, cnt_ref):
    i = pl.program_id(0)

    @pl.when(i == 0)
    def _():
        gsum_ref[...] = jnp.zeros_like(gsum_ref)
        cnt_ref[...] = jnp.zeros_like(cnt_ref)

    gids = lax.broadcasted_iota(jnp.int32, (NUM_GRAPHS, BN), 0)
    onehot = (gids == b_ref[...]).astype(jnp.float32)
    gsum_ref[...] += jnp.dot(onehot, h_ref[...],
                             preferred_element_type=jnp.float32)
    cnt_ref[...] += jnp.sum(onehot, axis=1, keepdims=True)

    @pl.when(i == N_BLOCKS - 1)
    def _():
        feats = gsum_ref[...] / jnp.maximum(cnt_ref[...], 1.0)
        z = _silu(jnp.dot(feats, wf1_ref[...],
                          preferred_element_type=jnp.float32) + bf1_ref[...])
        out_ref[...] = jnp.dot(z, wf2_ref[...],
                               preferred_element_type=jnp.float32) + bf2_ref[...]


def _pool_head(h, batch_row, wf1, bf1, wf2, bf2):
    full = lambda r, c: pl.BlockSpec((r, c), lambda i: (0, 0))
    return pl.pallas_call(
        _pool_body,
        grid=(N_BLOCKS,),
        in_specs=[
            pl.BlockSpec((BN, NF), lambda i: (i, 0)),
            pl.BlockSpec((1, BN), lambda i: (0, i)),
            full(NF, 128), full(1, 128), full(128, 1), full(1, 1),
        ],
        out_specs=full(NUM_GRAPHS, 1),
        out_shape=jax.ShapeDtypeStruct((NUM_GRAPHS, 1), jnp.float32),
        scratch_shapes=[
            pltpu.VMEM((NUM_GRAPHS, NF), jnp.float32),
            pltpu.VMEM((NUM_GRAPHS, 1), jnp.float32),
        ],
    )(h, batch_row, wf1, bf1, wf2, bf2)


def kernel(x, edge_index, sh, edge_attr, batch, W_atom, b_atom, W_sh1, b_sh1,
           W_sh2, b_sh2, W_r1, b_r1, W_r2, b_r2, Wm, bm, We, be, Wu, bu,
           W_f1, b_f1, W_f2, b_f2):
    f32 = jnp.float32
    row = lambda v: v.reshape(1, -1)

    npad = E_PAD - E
    pad_ar = jnp.arange(npad, dtype=jnp.int32)
    pad_src = (pad_ar * 17) % N
    pad_dst = N + pad_ar % (N_PAD - N)
    x_p = jnp.pad(x, ((0, N_PAD - N), (0, 0)))
    repack = lambda v: v.reshape(E_BLOCKS, 4, BE // 4).transpose(
        0, 2, 1).reshape(E_PAD // BLK, BLK)
    src_p = repack(jnp.concatenate([edge_index[0], pad_src]))
    dst_p = repack(jnp.concatenate([edge_index[1], pad_dst]))
    sh_p = jnp.pad(sh, ((0, npad), (0, 0)))
    ea_p = jnp.pad(edge_attr, (0, npad), constant_values=1.0).reshape(E_PAD, 1)
    batch_p = jnp.pad(batch, (0, N_PAD - N),
                      constant_values=NUM_GRAPHS).reshape(1, N_PAD)
    centers = jnp.linspace(0.125, 1.4, RBF_BINS).astype(f32)
    gam = (1.0 / (centers[1] - centers[0])).reshape(1, 1)
    cen = centers.reshape(1, RBF_BINS)
    we_cat = jnp.concatenate([We[0], We[1], We[2]], axis=1)
    be_cat = jnp.concatenate([be[0], be[1], be[2]]).reshape(1, 3 * NF)

    h, hm = _embed(x_p, W_atom, row(b_atom), Wm[0], row(bm[0]))
    g0, g1, g2 = _gates(sh_p, ea_p, cen, gam, W_r1, row(b_r1), W_r2,
                        row(b_r2), W_sh1, row(b_sh1), W_sh2, row(b_sh2),
                        we_cat, be_cat)
    gates = [g.reshape(E_PAD // BLK, BLK // 4, 128) for g in (g0, g1, g2)]

    for i in range(NUM_LAYERS):
        p0, p1 = _sc_layer(hm, src_p, dst_p, gates[i])
        last = i == NUM_LAYERS - 1
        if last:
            h = _update(h, p0, p1, Wu[i], row(bu[i]), None, None, True)
        else:
            h, hm = _update(h, p0, p1, Wu[i], row(bu[i]), Wm[i + 1],
                            row(bm[i + 1]), False)

    out = _pool_head(h, batch_p, W_f1, row(b_f1), W_f2, row(b_f2))
    return out[:, 0]
